# baseline (speedup 1.0000x reference)
import functools

import jax
import jax.numpy as jnp
import numpy as np
from jax.experimental import pallas as pl

NUM_PATCHES = 1024
D_MODEL = 768
MASK_TOKEN = -100.0
ROW_BLOCK = 128
GRID = NUM_PATCHES // ROW_BLOCK


def _compute_mask_np() -> np.ndarray:
    k = jax.random.key(42)
    idx = np.asarray(
        jax.random.uniform(k, (768,), minval=0.0, maxval=float(NUM_PATCHES))
    ).astype(np.int32)
    m = np.ones((NUM_PATCHES,), dtype=bool)
    m[idx] = False
    return m


_MASK_NP = _compute_mask_np()


def _select_body(mask_ref, x_ref, out_ref):
    out_ref[...] = jnp.where(mask_ref[...] != 0.0, x_ref[...], MASK_TOKEN)


@functools.partial(jax.jit, static_argnames=("interpret",))
def kernel(x, *, interpret=False):
    maskf = jnp.asarray(_MASK_NP[:, None], dtype=jnp.float32)
    patched = pl.pallas_call(
        _select_body,
        grid=(GRID,),
        in_specs=[
            pl.BlockSpec((ROW_BLOCK, 1), lambda i: (i, 0)),
            pl.BlockSpec((ROW_BLOCK, D_MODEL), lambda i: (i, 0)),
        ],
        out_specs=pl.BlockSpec((ROW_BLOCK, D_MODEL), lambda i: (i, 0)),
        out_shape=jax.ShapeDtypeStruct((NUM_PATCHES, D_MODEL), jnp.float32),
        interpret=interpret,
    )(maskf, x)
    return patched, jnp.asarray(_MASK_NP)
